# Initial kernel scaffold; baseline (speedup 1.0000x reference)
import jax
import jax.numpy as jnp
from jax.experimental import pallas as pl


def kernel(x, params, padj, fadj):
    raise NotImplementedError("write your pallas kernel here")


# speedup vs baseline: 4.7006x; 4.7006x over previous
import functools

import jax
import jax.numpy as jnp
from jax import lax
from jax.experimental import pallas as pl
from jax.experimental.pallas import tpu as pltpu
from jax.experimental.pallas import tpu_sc as plsc

_N = 10000
_E = 320000
_F = 128
_H = 64
_BR = 1000
_NBLK = _N // _BR

_NC = 2
_NS = 16
_NW = _NC * _NS
_EPW = _E // _NW
_CHUNK = 80
_NCHUNK = _EPW // _CHUNK
_NPAD = 10240
_RPS = _NPAD // _NS


def _spmm128(u, src, dst, zeros):
  mesh = plsc.VectorSubcoreMesh(core_axis_name="c", subcore_axis_name="s")

  @functools.partial(
      pl.kernel,
      out_type=jax.ShapeDtypeStruct((_NC, _NPAD, _F), jnp.float32),
      mesh=mesh,
      scratch_types=[
          pltpu.VMEM((_CHUNK,), jnp.int32),
          pltpu.VMEM((_CHUNK,), jnp.int32),
          pltpu.VMEM((_CHUNK, _F), jnp.float32),
          pltpu.VMEM_SHARED((_NPAD, _F), jnp.float32),
          pltpu.SemaphoreType.DMA,
      ],
  )
  def k(u_hbm, src_hbm, dst_hbm, z_hbm, out_hbm, src_v, dst_v, rows_v, acc_sh,
        sem):
    cid = lax.axis_index("c")
    sid = lax.axis_index("s")
    wid = sid * _NC + cid
    r0 = sid * _RPS
    pltpu.sync_copy(z_hbm.at[pl.ds(r0, _RPS)], acc_sh.at[pl.ds(r0, _RPS)])
    plsc.subcore_barrier()
    base = wid * _EPW

    def body(i, carry):
      off = base + i * _CHUNK
      pltpu.sync_copy(src_hbm.at[pl.ds(off, _CHUNK)], src_v)
      pltpu.sync_copy(dst_hbm.at[pl.ds(off, _CHUNK)], dst_v)
      pltpu.async_copy(u_hbm.at[src_v], rows_v, sem).wait()
      pltpu.sync_copy(rows_v, acc_sh.at[dst_v], add=True)
      return carry

    lax.fori_loop(0, _NCHUNK, body, 0)
    plsc.subcore_barrier()
    pltpu.sync_copy(acc_sh.at[pl.ds(r0, _RPS)],
                    out_hbm.at[cid, pl.ds(r0, _RPS)])

  return k(u, src, dst, zeros)


def _xproj(x, w):
  kdim = w.shape[1]

  def body(x_ref, w_ref, o_ref):
    o_ref[...] = jnp.dot(x_ref[...], w_ref[...],
                         preferred_element_type=jnp.float32)

  return pl.pallas_call(
      body,
      grid=(_NBLK,),
      in_specs=[
          pl.BlockSpec((_BR, _F), lambda i: (i, 0)),
          pl.BlockSpec((_F, kdim), lambda i: (0, 0)),
      ],
      out_specs=pl.BlockSpec((_BR, kdim), lambda i: (i, 0)),
      out_shape=jax.ShapeDtypeStruct((_N, kdim), jnp.float32),
  )(x, w)


def _combine_stats(p, bias):
  def body(p_ref, b_ref, h_ref, mv_ref, sv_ref, acc):
    i = pl.program_id(0)
    h = p_ref[0] + p_ref[1] + b_ref[...]
    h_ref[...] = h

    @pl.when(i == 0)
    def _():
      acc[...] = jnp.zeros_like(acc)

    acc[...] += jnp.concatenate(
        [jnp.sum(h, axis=0, keepdims=True),
         jnp.sum(h * h, axis=0, keepdims=True)], axis=0)

    @pl.when(i == _NBLK - 1)
    def _():
      a = acc[...]
      mean = a[0:1] / _N
      var = a[1:2] - _N * mean * mean
      lanes = lax.broadcasted_iota(jnp.int32, (1, _F), 1)
      left = lanes < _H
      suma = jnp.sum(jnp.where(left, var, 0.0))
      sumb = jnp.sum(jnp.where(left, 0.0, var))
      sfa = lax.rsqrt(1e-6 + suma / _N)
      sfb = lax.rsqrt(1e-6 + sumb / _N)
      mv_ref[...] = mean
      sv_ref[...] = jnp.where(left, sfa, sfb)

  return pl.pallas_call(
      body,
      grid=(_NBLK,),
      in_specs=[
          pl.BlockSpec((2, _BR, _F), lambda i: (0, i, 0)),
          pl.BlockSpec((1, _F), lambda i: (0, 0)),
      ],
      out_specs=[
          pl.BlockSpec((_BR, _F), lambda i: (i, 0)),
          pl.BlockSpec((1, _F), lambda i: (0, 0)),
          pl.BlockSpec((1, _F), lambda i: (0, 0)),
      ],
      out_shape=[
          jax.ShapeDtypeStruct((_N, _F), jnp.float32),
          jax.ShapeDtypeStruct((1, _F), jnp.float32),
          jax.ShapeDtypeStruct((1, _F), jnp.float32),
      ],
      scratch_shapes=[pltpu.VMEM((2, _F), jnp.float32)],
  )(p, bias.reshape(1, _F))


def _layer_mm(xp, h, mv, sv, wb):
  def body(xp_ref, h_ref, mv_ref, sv_ref, wb_ref, u_ref, b_ref):
    blk = jnp.tanh((h_ref[...] - mv_ref[...]) * sv_ref[...])
    b_ref[...] = blk
    u_ref[...] = xp_ref[...] + jnp.dot(blk, wb_ref[...],
                                       preferred_element_type=jnp.float32)

  return pl.pallas_call(
      body,
      grid=(_NBLK,),
      in_specs=[
          pl.BlockSpec((_BR, _F), lambda i: (i, 0)),
          pl.BlockSpec((_BR, _F), lambda i: (i, 0)),
          pl.BlockSpec((1, _F), lambda i: (0, 0)),
          pl.BlockSpec((1, _F), lambda i: (0, 0)),
          pl.BlockSpec((_F, _F), lambda i: (0, 0)),
      ],
      out_specs=[
          pl.BlockSpec((_BR, _F), lambda i: (i, 0)),
          pl.BlockSpec((_BR, _F), lambda i: (i, 0)),
      ],
      out_shape=[
          jax.ShapeDtypeStruct((_N, _F), jnp.float32),
          jax.ShapeDtypeStruct((_N, _F), jnp.float32),
      ],
  )(xp, h, mv, sv, wb)


def _out_mm(xp, b0, wb0, h1, mv, sv, wb1):
  def body(xp_ref, b0_ref, wb0_ref, h1_ref, mv_ref, sv_ref, wb1_ref, u_ref):
    b1 = jnp.tanh((h1_ref[...] - mv_ref[...]) * sv_ref[...])
    u_ref[...] = (xp_ref[...] +
                  jnp.dot(b0_ref[...], wb0_ref[...],
                          preferred_element_type=jnp.float32) +
                  jnp.dot(b1, wb1_ref[...],
                          preferred_element_type=jnp.float32))

  return pl.pallas_call(
      body,
      grid=(_NBLK,),
      in_specs=[
          pl.BlockSpec((_BR, _F), lambda i: (i, 0)),
          pl.BlockSpec((_BR, _F), lambda i: (i, 0)),
          pl.BlockSpec((_F, _F), lambda i: (0, 0)),
          pl.BlockSpec((_BR, _F), lambda i: (i, 0)),
          pl.BlockSpec((1, _F), lambda i: (0, 0)),
          pl.BlockSpec((1, _F), lambda i: (0, 0)),
          pl.BlockSpec((_F, _F), lambda i: (0, 0)),
      ],
      out_specs=pl.BlockSpec((_BR, _F), lambda i: (i, 0)),
      out_shape=jax.ShapeDtypeStruct((_N, _F), jnp.float32),
  )(xp, b0, wb0, h1, mv, sv, wb1)


def _rownorm(p, bias):
  def body(p_ref, b_ref, o_ref):
    h = p_ref[0] + p_ref[1] + b_ref[...]
    lanes = lax.broadcasted_iota(jnp.int32, (_BR, _F), 1)
    left = lanes < _H
    sq = h * h
    na = jnp.sqrt(jnp.sum(jnp.where(left, sq, 0.0), axis=1, keepdims=True))
    nb = jnp.sqrt(jnp.sum(jnp.where(left, 0.0, sq), axis=1, keepdims=True))
    n = jnp.where(left, jnp.maximum(na, 1e-12), jnp.maximum(nb, 1e-12))
    o_ref[...] = h / n

  return pl.pallas_call(
      body,
      grid=(_NBLK,),
      in_specs=[
          pl.BlockSpec((2, _BR, _F), lambda i: (0, i, 0)),
          pl.BlockSpec((1, _F), lambda i: (0, 0)),
      ],
      out_specs=pl.BlockSpec((_BR, _F), lambda i: (i, 0)),
      out_shape=jax.ShapeDtypeStruct((_N, _F), jnp.float32),
  )(p, bias.reshape(1, _F))


def _fuse(op, of, aw1, ab1, aw2, mw, mb):
  def body(op_ref, of_ref, aw1_ref, ab1_ref, aw2_ref, mw_ref, mb_ref,
           out_ref, beta_ref):
    vop = op_ref[...]
    vof = of_ref[...]
    e1 = vop[:, :_H]
    c1 = vop[:, _H:]
    e2 = vof[:, :_H]
    c2 = vof[:, _H:]
    xc = (c1 + c2) * 0.5
    aw1 = aw1_ref[...]
    ab1 = ab1_ref[...]
    aw2 = aw2_ref[...]

    def att(v):
      t = jnp.tanh(jnp.dot(v, aw1, preferred_element_type=jnp.float32) + ab1)
      return jnp.sum(t * aw2, axis=1, keepdims=True)

    w1 = att(e1)
    w2 = att(e2)
    w3 = att(xc)
    m = jnp.maximum(jnp.maximum(w1, w2), w3)
    x1 = jnp.exp(w1 - m)
    x2 = jnp.exp(w2 - m)
    x3 = jnp.exp(w3 - m)
    s = x1 + x2 + x3
    emb = (x1 * e1 + x2 * e2 + x3 * xc) / s
    cols = lax.broadcasted_iota(jnp.int32, (_BR, 3), 1)
    beta_ref[...] = jnp.where(cols == 0, x1, jnp.where(cols == 1, x2, x3)) / s
    logits = jnp.dot(emb, mw_ref[...],
                     preferred_element_type=jnp.float32) + mb_ref[...]
    mx = jnp.max(logits, axis=1, keepdims=True)
    ex = jnp.exp(logits - mx)
    out_ref[...] = ex / jnp.sum(ex, axis=1, keepdims=True)

  return pl.pallas_call(
      body,
      grid=(_NBLK,),
      in_specs=[
          pl.BlockSpec((_BR, _F), lambda i: (i, 0)),
          pl.BlockSpec((_BR, _F), lambda i: (i, 0)),
          pl.BlockSpec((_H, 2), lambda i: (0, 0)),
          pl.BlockSpec((1, 2), lambda i: (0, 0)),
          pl.BlockSpec((1, 2), lambda i: (0, 0)),
          pl.BlockSpec((_H, 16), lambda i: (0, 0)),
          pl.BlockSpec((1, 16), lambda i: (0, 0)),
      ],
      out_specs=[
          pl.BlockSpec((_BR, 16), lambda i: (i, 0)),
          pl.BlockSpec((_BR, 3), lambda i: (i, 0)),
      ],
      out_shape=[
          jax.ShapeDtypeStruct((_N, 16), jnp.float32),
          jax.ShapeDtypeStruct((_N, 3), jnp.float32),
      ],
  )(op, of, aw1, ab1, aw2, mw, mb)


def _blockdiag(a, b):
  top = jnp.concatenate([a, jnp.zeros((a.shape[0], b.shape[1]), jnp.float32)],
                        axis=1)
  bot = jnp.concatenate([jnp.zeros((b.shape[0], a.shape[1]), jnp.float32), b],
                        axis=1)
  return jnp.concatenate([top, bot], axis=0)


def _pair_weights(pa, pb):
  w0 = jnp.concatenate([pa["ws"][0], pb["ws"][0]], axis=1)
  w1x = jnp.concatenate([pa["ws"][1][:_F], pb["ws"][1][:_F]], axis=1)
  w1b = _blockdiag(pa["ws"][1][_F:], pb["ws"][1][_F:])
  wox = jnp.concatenate([pa["w_out"][:_F], pb["w_out"][:_F]], axis=1)
  wob0 = _blockdiag(pa["w_out"][_F:_F + _H], pb["w_out"][_F:_F + _H])
  wob1 = _blockdiag(pa["w_out"][_F + _H:], pb["w_out"][_F + _H:])
  b0 = jnp.concatenate([pa["bs"][0], pb["bs"][0]])
  b1 = jnp.concatenate([pa["bs"][1], pb["bs"][1]])
  bo = jnp.concatenate([pa["b_out"], pb["b_out"]])
  return w0, w1x, w1b, wox, wob0, wob1, b0, b1, bo


def kernel(x, params, padj, fadj):
  wp = _pair_weights(params["sgcn1"], params["cgcn"])
  wf = _pair_weights(params["sgcn2"], params["cgcn"])
  wall = jnp.concatenate([wp[0], wp[1], wp[3], wf[0], wf[1], wf[3]], axis=1)
  xp = _xproj(x, wall)
  zeros = jnp.zeros((_NPAD, _F), jnp.float32)

  def run(adj, w, xo):
    _, _, w1b, _, wob0, wob1, b0, b1, bo = w
    src = adj[0]
    dst = adj[1]
    u0 = xp[:, xo:xo + _F]
    p0 = _spmm128(u0, src, dst, zeros)
    h0, mv0, sv0 = _combine_stats(p0, b0)
    u1, blk0 = _layer_mm(xp[:, xo + _F:xo + 2 * _F], h0, mv0, sv0, w1b)
    p1 = _spmm128(u1, src, dst, zeros)
    h1, mv1, sv1 = _combine_stats(p1, b1)
    uo = _out_mm(xp[:, xo + 2 * _F:xo + 3 * _F], blk0, wob0, h1, mv1, sv1,
                 wob1)
    po = _spmm128(uo, src, dst, zeros)
    return _rownorm(po, bo)

  o_p = run(padj, wp, 0)
  o_f = run(fadj, wf, 3 * _F)
  out, beta = _fuse(o_p, o_f, params["att_w1"],
                    params["att_b1"].reshape(1, 2),
                    params["att_w2"].reshape(1, 2), params["mlp_w"],
                    params["mlp_b"].reshape(1, 16))
  emb1 = o_p[:, :_H]
  com1 = o_p[:, _H:]
  emb2 = o_f[:, :_H]
  com2 = o_f[:, _H:]
  return (out, jnp.zeros((1,), jnp.float32), beta.reshape(_N, 3, 1), emb1,
          com1, com2, emb2)
